# baseline (speedup 1.0000x reference)
import jax
import jax.numpy as jnp
from jax.experimental import pallas as pl
from jax.experimental.pallas import tpu as pltpu
from jax.experimental.pallas import tpu_sc as plsc


def _sc_gather(v2, ii2, gw):
    b = ii2.shape[1]
    r = v2.shape[1]
    mesh = plsc.VectorSubcoreMesh(core_axis_name="core", subcore_axis_name="subcore")

    @pl.kernel(out_type=jax.ShapeDtypeStruct((b, r), v2.dtype), mesh=mesh)
    def gather_kernel(v_hbm, i_hbm, o_hbm):
        def body(i_vmem, o_vmem):
            pltpu.sync_copy(v_hbm.at[i_vmem.at[0]], o_vmem)

        pltpu.emit_pipeline(
            body,
            grid=(b // gw,),
            in_specs=[pl.BlockSpec((1, gw), index_map=lambda i: (0, i))],
            out_specs=[pl.BlockSpec((gw, r), index_map=lambda i: (i, 0))],
            core_axis_name=("core", "subcore"),
            dimension_semantics=(pltpu.PARALLEL,),
        )(i_hbm, o_hbm)

    return gather_kernel(v2, ii2)


def _tc_body(vg_ref, utf_ref, o_ref, *, d, n, dn_out):
    xs = []
    for dd in range(d):
        xd = vg_ref[:, dd * n:(dd + 1) * n]
        xs.append(jnp.concatenate([xd, xd], axis=1))
    for u in range(dn_out // 128):
        sl = pl.ds(u * 128, 128)
        acc = xs[0] * utf_ref[0, sl][None, :]
        for dd in range(1, d):
            acc += xs[dd] * utf_ref[dd, sl][None, :]
        o_ref[:, sl] = acc


def _tc_einsum(vg2, utf, bB):
    b, dn_in = vg2.shape
    d, dn_out = utf.shape
    n = dn_in // d
    import functools
    body = functools.partial(_tc_body, d=d, n=n, dn_out=dn_out)
    return pl.pallas_call(
        body,
        grid=(b // bB,),
        in_specs=[
            pl.BlockSpec((bB, dn_in), lambda i: (i, 0)),
            pl.BlockSpec((d, dn_out), lambda i: (0, 0)),
        ],
        out_specs=pl.BlockSpec((bB, dn_out), lambda i: (i, 0)),
        out_shape=jax.ShapeDtypeStruct((b, dn_out), vg2.dtype),
        compiler_params=pltpu.CompilerParams(
            dimension_semantics=("arbitrary",),
        ),
    )(vg2, utf)


def kernel(ii, C, V, Us):
    del C
    nN, d, n = V.shape
    _, D, _ = Us.shape
    b = ii.shape[0]
    rpe = (d * n) // 128
    v2 = V.reshape(nN * rpe, 128)
    jj = (ii[:, None].astype(jnp.int32) * rpe
          + jnp.arange(rpe, dtype=jnp.int32)[None, :]).reshape(1, b * rpe)
    vg2 = _sc_gather(v2, jj, gw=128).reshape(b, d * n)
    utf = jnp.transpose(Us, (2, 1, 0)).reshape(d, D * n)
    y = _tc_einsum(vg2, utf, bB=256)
    return y.reshape(b, D, n)
